# baseline (speedup 1.0000x reference)
import functools

import jax
import jax.numpy as jnp
from jax import lax
from jax.experimental import pallas as pl
from jax.experimental.pallas import tpu as pltpu
from jax.experimental.pallas import tpu_sc as plsc

_NC = 2


@functools.partial(jax.jit, static_argnums=(2, 3))
def _sc_row_copy(tbl, idx, n_rows, d):
    b_per_c = n_rows // _NC
    mesh = plsc.ScalarSubcoreMesh(axis_name="c", num_cores=_NC)

    @functools.partial(
        pl.kernel,
        out_type=jax.ShapeDtypeStruct((n_rows, d), jnp.float32),
        mesh=mesh,
        scratch_types=[
            pltpu.SMEM((b_per_c,), jnp.int32),
            pltpu.SemaphoreType.DMA,
        ],
    )
    def k(tbl_hbm, idx_hbm, out_hbm, idx_s, sem):
        base = lax.axis_index("c") * b_per_c
        pltpu.sync_copy(idx_hbm.at[pl.ds(base, b_per_c)], idx_s)

        @pl.loop(0, b_per_c)
        def _(i):
            r = idx_s[i]
            pltpu.async_copy(
                tbl_hbm.at[pl.ds(r, 1)], out_hbm.at[pl.ds(base + i, 1)], sem)

        @pl.loop(0, b_per_c)
        def _(i):
            pltpu.make_async_copy(
                tbl_hbm.at[pl.ds(0, 1)], out_hbm.at[pl.ds(0, 1)], sem).wait()

    return k(tbl, idx)


def kernel(prefix, embedding_table):
    V, D = embedding_table.shape
    B = prefix.size
    idx = prefix.reshape(-1).astype(jnp.int32)
    out = _sc_row_copy(embedding_table, idx, B, D)
    return out.reshape(*prefix.shape, D)


# speedup vs baseline: 30.5490x; 30.5490x over previous
import functools

import jax
import jax.numpy as jnp
from jax import lax
from jax.experimental import pallas as pl
from jax.experimental.pallas import tpu as pltpu
from jax.experimental.pallas import tpu_sc as plsc

_NC = 2
_NSLOT = 8


@functools.partial(jax.jit, static_argnums=(2, 3))
def _sc_row_copy(tbl, idx, n_rows, d):
    b_per_c = n_rows // _NC
    n_steps = b_per_c // _NSLOT
    mesh = plsc.ScalarSubcoreMesh(axis_name="c", num_cores=_NC)

    @functools.partial(
        pl.kernel,
        out_type=jax.ShapeDtypeStruct((n_rows, d), jnp.float32),
        mesh=mesh,
        scratch_types=[
            pltpu.SMEM((b_per_c,), jnp.int32),
            pltpu.VMEM_SHARED((_NSLOT, d), jnp.float32),
            [pltpu.SemaphoreType.DMA] * _NSLOT,
            [pltpu.SemaphoreType.DMA] * _NSLOT,
        ],
    )
    def k(tbl_hbm, idx_hbm, out_hbm, idx_s, rows, gsem, ssem):
        base = lax.axis_index("c") * b_per_c
        pltpu.sync_copy(idx_hbm.at[pl.ds(base, b_per_c)], idx_s)

        def gather(i, t):
            return pltpu.make_async_copy(
                tbl_hbm.at[pl.ds(idx_s[i], 1)], rows.at[pl.ds(t, 1)], gsem[t])

        def scatter(i, t):
            return pltpu.make_async_copy(
                rows.at[pl.ds(t, 1)], out_hbm.at[pl.ds(base + i, 1)], ssem[t])

        for t in range(_NSLOT):
            gather(t, t).start()
        for t in range(_NSLOT):
            gather(t, t).wait()
            scatter(t, t).start()

        @pl.loop(1, n_steps)
        def _(j):
            b0 = j * _NSLOT
            for t in range(_NSLOT):
                scatter(0, t).wait()
                gather(b0 + t, t).start()
            for t in range(_NSLOT):
                gather(0, t).wait()
                scatter(b0 + t, t).start()

        for t in range(_NSLOT):
            scatter(0, t).wait()

    return k(tbl, idx)


def kernel(prefix, embedding_table):
    V, D = embedding_table.shape
    B = prefix.size
    idx = prefix.reshape(-1).astype(jnp.int32)
    out = _sc_row_copy(embedding_table, idx, B, D)
    return out.reshape(*prefix.shape, D)


# speedup vs baseline: 36.7961x; 1.2045x over previous
import functools

import jax
import jax.numpy as jnp
from jax import lax
from jax.experimental import pallas as pl
from jax.experimental.pallas import tpu as pltpu
from jax.experimental.pallas import tpu_sc as plsc

_NC = 2
_G = 8
_K = 4


@functools.partial(jax.jit, static_argnums=(2, 3))
def _sc_row_copy(tbl, idx, n_rows, d):
    b_per_c = n_rows // _NC
    n_grp = b_per_c // _G
    mesh = plsc.ScalarSubcoreMesh(axis_name="c", num_cores=_NC)

    @functools.partial(
        pl.kernel,
        out_type=jax.ShapeDtypeStruct((n_rows, d), jnp.float32),
        mesh=mesh,
        scratch_types=[
            pltpu.SMEM((b_per_c,), jnp.int32),
            pltpu.VMEM_SHARED((_K * _G, d), jnp.float32),
            [pltpu.SemaphoreType.DMA] * _K,
            [pltpu.SemaphoreType.DMA] * _K,
        ],
    )
    def k(tbl_hbm, idx_hbm, out_hbm, idx_s, rows, gsem, ssem):
        base = lax.axis_index("c") * b_per_c
        pltpu.sync_copy(idx_hbm.at[pl.ds(base, b_per_c)], idx_s)

        def gather_grp(g, t):
            for u in range(_G):
                pltpu.make_async_copy(
                    tbl_hbm.at[pl.ds(idx_s[g * _G + u], 1)],
                    rows.at[pl.ds(t * _G + u, 1)], gsem[t]).start()

        def gather_wait(t):
            pltpu.make_async_copy(
                tbl_hbm.at[pl.ds(0, _G)],
                rows.at[pl.ds(t * _G, _G)], gsem[t]).wait()

        def scatter_grp(g, t):
            return pltpu.make_async_copy(
                rows.at[pl.ds(t * _G, _G)],
                out_hbm.at[pl.ds(base + g * _G, _G)], ssem[t])

        for t in range(_K):
            gather_grp(t, t)
        for t in range(_K):
            gather_wait(t)
            scatter_grp(t, t).start()

        @pl.loop(1, n_grp // _K)
        def _(j):
            g0 = j * _K
            for t in range(_K):
                scatter_grp(0, t).wait()
                gather_grp(g0 + t, t)
            for t in range(_K):
                gather_wait(t)
                scatter_grp(g0 + t, t).start()

        for t in range(_K):
            scatter_grp(0, t).wait()

    return k(tbl, idx)


def kernel(prefix, embedding_table):
    V, D = embedding_table.shape
    B = prefix.size
    idx = prefix.reshape(-1).astype(jnp.int32)
    out = _sc_row_copy(embedding_table, idx, B, D)
    return out.reshape(*prefix.shape, D)
